# baseline (speedup 1.0000x reference)
import functools

import jax
import jax.numpy as jnp
from jax import lax
from jax.experimental import pallas as pl
from jax.experimental.pallas import tpu as pltpu
from jax.experimental.pallas import tpu_sc as plsc

B = 4
SEQ = 4096
D = 2048
NUM_BUCKETS = 64
MAX_SKIPPED_FLOPS = float(12 * (12 * D * D * SEQ))

R = B * SEQ
NC = 2
NS = 16
NW = NC * NS
RPW = R // NW
C = 16
NCHUNK = RPW // C
CU = 8

_mesh = plsc.VectorSubcoreMesh(core_axis_name="c", subcore_axis_name="s")


@functools.partial(
    pl.kernel,
    mesh=_mesh,
    out_type=jax.ShapeDtypeStruct((R, D), jnp.float32),
    scratch_types=[
        pltpu.VMEM((C,), jnp.int32),
        pltpu.VMEM((C,), jnp.float32),
        pltpu.VMEM((C, D), jnp.float32),
        pltpu.VMEM((C, D), jnp.float32),
        pltpu.SemaphoreType.DMA,
        pltpu.SemaphoreType.DMA,
    ],
)
def _sc_step_encoding(x_hbm, csf_hbm, table_hbm, out_hbm,
                      idx_v, csf_v, xbuf, embbuf, xsem, gsem):
    wid = lax.axis_index("s") * NC + lax.axis_index("c")
    base = wid * RPW

    def chunk_body(ci):
        row0 = base + ci * C
        xcopy = pltpu.async_copy(x_hbm.at[pl.ds(row0, C)], xbuf, xsem)

        pltpu.sync_copy(csf_hbm.at[pl.ds(row0, C)], csf_v)
        v = csf_v[...]
        bkt = ((v / MAX_SKIPPED_FLOPS) * float(NUM_BUCKETS)).astype(jnp.int32)
        bkt = jnp.minimum(jnp.maximum(bkt, 0), NUM_BUCKETS - 1)
        idx_v[...] = bkt

        pltpu.async_copy(table_hbm.at[idx_v], embbuf, gsem).wait()
        xcopy.wait()

        def row_body(r):
            def col_body(k):
                for u in range(CU):
                    off = k * (16 * CU) + u * 16
                    ev = embbuf[r, pl.ds(off, 16)]
                    plsc.addupdate(xbuf.at[r, pl.ds(off, 16)], ev)
            pl.loop(0, D // (16 * CU))(col_body)
        pl.loop(0, C)(row_body)

        pltpu.sync_copy(xbuf, out_hbm.at[pl.ds(row0, C)])

    pl.loop(0, NCHUNK)(chunk_body)


def kernel(x, cumulative_skipped_flops, step_embeddings_weight):
    out = _sc_step_encoding(
        x.reshape(R, D),
        cumulative_skipped_flops.reshape(R),
        step_embeddings_weight,
    )
    return out.reshape(B, SEQ, D)


# speedup vs baseline: 1.4975x; 1.4975x over previous
import functools

import jax
import jax.numpy as jnp
from jax import lax
from jax.experimental import pallas as pl
from jax.experimental.pallas import tpu as pltpu
from jax.experimental.pallas import tpu_sc as plsc

B = 4
SEQ = 4096
D = 2048
NUM_BUCKETS = 64
MAX_SKIPPED_FLOPS = float(12 * (12 * D * D * SEQ))

R = B * SEQ
NC = 2
NS = 16
NW = NC * NS
RPW = R // NW
C = 8
NCH = RPW // C

_mesh = plsc.VectorSubcoreMesh(core_axis_name="c", subcore_axis_name="s")


@functools.partial(
    pl.kernel,
    mesh=_mesh,
    out_type=jax.ShapeDtypeStruct((R, D), jnp.float32),
    scratch_types=[
        pltpu.VMEM((RPW,), jnp.float32),
        pltpu.VMEM((RPW,), jnp.int32),
        pltpu.VMEM((C, D), jnp.float32),
        pltpu.VMEM((C, D), jnp.float32),
        pltpu.VMEM((C, D), jnp.float32),
        pltpu.VMEM((C, D), jnp.float32),
        pltpu.VMEM((C, D), jnp.float32),
        pltpu.VMEM((C, D), jnp.float32),
        pltpu.SemaphoreType.DMA,
        pltpu.SemaphoreType.DMA,
        pltpu.SemaphoreType.DMA,
        pltpu.SemaphoreType.DMA,
        pltpu.SemaphoreType.DMA,
        pltpu.SemaphoreType.DMA,
        pltpu.SemaphoreType.DMA,
        pltpu.SemaphoreType.DMA,
        pltpu.SemaphoreType.DMA,
        pltpu.SemaphoreType.DMA,
    ],
)
def _sc_step_encoding(x_hbm, csf_hbm, table_hbm, out_hbm,
                      csf_v, idx_v, xb0, xb1, eb0, eb1, eb2, eb3,
                      xs0, xs1, gs0, gs1, gs2, gs3, os0, os1, os2, os3):
    xbufs = (xb0, xb1)
    ebufs = (eb0, eb1, eb2, eb3)
    xsems = (xs0, xs1)
    gsems = (gs0, gs1, gs2, gs3)
    osems = (os0, os1, os2, os3)

    wid = lax.axis_index("s") * NC + lax.axis_index("c")
    base = wid * RPW

    pltpu.sync_copy(csf_hbm.at[pl.ds(base, RPW)], csf_v)

    def idx_body(j):
        v = csf_v[pl.ds(j * 16, 16)]
        bkt = ((v / MAX_SKIPPED_FLOPS) * float(NUM_BUCKETS)).astype(jnp.int32)
        idx_v[pl.ds(j * 16, 16)] = jnp.minimum(
            jnp.maximum(bkt, 0), NUM_BUCKETS - 1)
    pl.loop(0, RPW // 16)(idx_body)

    def rows(ci):
        return pl.ds(base + ci * C, C)

    def start_x(ci, s):
        pltpu.async_copy(x_hbm.at[rows(ci)], xbufs[s], xsems[s])

    def wait_x(ci, s):
        pltpu.make_async_copy(x_hbm.at[rows(ci)], xbufs[s], xsems[s]).wait()

    def start_g(ci, s):
        pltpu.async_copy(
            table_hbm.at[idx_v.at[pl.ds(ci * C, C)]], ebufs[s], gsems[s])

    def wait_g(ci, s):
        pltpu.make_async_copy(
            table_hbm.at[idx_v.at[pl.ds(ci * C, C)]], ebufs[s],
            gsems[s]).wait()

    def start_o(ci, s):
        pltpu.async_copy(ebufs[s], out_hbm.at[rows(ci)], osems[s])

    def wait_o(ci, s):
        pltpu.make_async_copy(ebufs[s], out_hbm.at[rows(ci)], osems[s]).wait()

    def add_chunk(xs, es):
        xb = xbufs[xs]
        eb = ebufs[es]

        def row_body(r):
            def col_body(k):
                for u in range(8):
                    off = k * 128 + u * 16
                    xv = xb[r, pl.ds(off, 16)]
                    plsc.addupdate(eb.at[r, pl.ds(off, 16)], xv)
            pl.loop(0, D // 128)(col_body)
        pl.loop(0, C)(row_body)

    def proc(ci, u, first, last):
        xs, es = u % 2, u % 4
        wait_x(ci, xs)
        wait_g(ci, es)
        add_chunk(xs, es)
        start_o(ci, es)
        if not last:
            start_x(ci + 2, xs)
        if not first:
            wait_o(ci - 2, (es + 2) % 4)
        if not last:
            start_g(ci + 2, (es + 2) % 4)

    start_x(0, 0)
    start_x(1, 1)
    start_g(0, 0)
    start_g(1, 1)
    for u in range(4):
        proc(u, u, first=(u < 2), last=False)

    def quad_body(q):
        for u in range(4):
            proc(q * 4 + u, u, first=False, last=False)
    pl.loop(1, NCH // 4 - 1)(quad_body)

    for u in range(4):
        proc(NCH - 4 + u, u, first=False, last=(u >= 2))
    for u in (2, 3):
        wait_o(NCH - 4 + u, u)


def kernel(x, cumulative_skipped_flops, step_embeddings_weight):
    out = _sc_step_encoding(
        x.reshape(R, D),
        cumulative_skipped_flops.reshape(R),
        step_embeddings_weight,
    )
    return out.reshape(B, SEQ, D)


# speedup vs baseline: 2.3776x; 1.5877x over previous
import functools

import jax
import jax.numpy as jnp
from jax import lax
from jax.experimental import pallas as pl
from jax.experimental.pallas import tpu as pltpu
from jax.experimental.pallas import tpu_sc as plsc

B = 4
SEQ = 4096
D = 2048
NUM_BUCKETS = 64
MAX_SKIPPED_FLOPS = float(12 * (12 * D * D * SEQ))

R = B * SEQ
NC = 2
NS = 16
NW = NC * NS
RPW = R // NW
C = 8
NCH = RPW // C

_mesh = plsc.VectorSubcoreMesh(core_axis_name="c", subcore_axis_name="s")


@functools.partial(
    pl.kernel,
    mesh=_mesh,
    out_type=jax.ShapeDtypeStruct((R, D), jnp.float32),
    scratch_types=[
        pltpu.VMEM((RPW,), jnp.float32),
        pltpu.VMEM((RPW,), jnp.int32),
        pltpu.VMEM((C, D), jnp.float32),
        pltpu.VMEM((C, D), jnp.float32),
        pltpu.VMEM((C, D), jnp.float32),
        pltpu.VMEM((C, D), jnp.float32),
        pltpu.VMEM((C, D), jnp.float32),
        pltpu.VMEM((C, D), jnp.float32),
        pltpu.SemaphoreType.DMA,
        pltpu.SemaphoreType.DMA,
        pltpu.SemaphoreType.DMA,
        pltpu.SemaphoreType.DMA,
        pltpu.SemaphoreType.DMA,
        pltpu.SemaphoreType.DMA,
        pltpu.SemaphoreType.DMA,
        pltpu.SemaphoreType.DMA,
        pltpu.SemaphoreType.DMA,
        pltpu.SemaphoreType.DMA,
    ],
)
def _sc_step_encoding(x_hbm, csf_hbm, table_hbm, out_hbm,
                      csf_v, idx_v, xb0, xb1, eb0, eb1, eb2, eb3,
                      xs0, xs1, gs0, gs1, gs2, gs3, os0, os1, os2, os3):
    xbufs = (xb0, xb1)
    ebufs = (eb0, eb1, eb2, eb3)
    xsems = (xs0, xs1)
    gsems = (gs0, gs1, gs2, gs3)
    osems = (os0, os1, os2, os3)

    wid = lax.axis_index("s") * NC + lax.axis_index("c")
    base = wid * RPW

    pltpu.sync_copy(csf_hbm.at[pl.ds(base, RPW)], csf_v)

    def idx_body(j):
        v = csf_v[pl.ds(j * 16, 16)]
        bkt = ((v / MAX_SKIPPED_FLOPS) * float(NUM_BUCKETS)).astype(jnp.int32)
        idx_v[pl.ds(j * 16, 16)] = jnp.minimum(
            jnp.maximum(bkt, 0), NUM_BUCKETS - 1)
    pl.loop(0, RPW // 16)(idx_body)

    def rows(ci):
        return pl.ds(base + ci * C, C)

    def start_x(ci, s):
        pltpu.async_copy(x_hbm.at[rows(ci)], xbufs[s], xsems[s])

    def wait_x(ci, s):
        pltpu.make_async_copy(x_hbm.at[rows(ci)], xbufs[s], xsems[s]).wait()

    def start_g(ci, s):
        pltpu.async_copy(
            table_hbm.at[idx_v.at[pl.ds(ci * C, C)]], ebufs[s], gsems[s])

    def wait_g(ci, s):
        pltpu.make_async_copy(
            table_hbm.at[idx_v.at[pl.ds(ci * C, C)]], ebufs[s],
            gsems[s]).wait()

    def start_o(ci, s):
        pltpu.async_copy(ebufs[s], out_hbm.at[rows(ci)], osems[s])

    def wait_o(ci, s):
        pltpu.make_async_copy(ebufs[s], out_hbm.at[rows(ci)], osems[s]).wait()

    def add_chunk(xs, es):
        xb = xbufs[xs]
        eb = ebufs[es]

        def row_body(r):
            @functools.partial(plsc.parallel_loop, 0, D // 16, unroll=8)
            def col_body(k):
                off = k * 16
                xv = xb[r, pl.ds(off, 16)]
                plsc.addupdate(eb.at[r, pl.ds(off, 16)], xv)
        pl.loop(0, C)(row_body)

    def proc(ci, u, first, last):
        xs, es = u % 2, u % 4
        wait_x(ci, xs)
        wait_g(ci, es)
        add_chunk(xs, es)
        start_o(ci, es)
        if not last:
            start_x(ci + 2, xs)
        if not first:
            wait_o(ci - 2, (es + 2) % 4)
        if not last:
            start_g(ci + 2, (es + 2) % 4)

    start_x(0, 0)
    start_x(1, 1)
    start_g(0, 0)
    start_g(1, 1)
    for u in range(4):
        proc(u, u, first=(u < 2), last=False)

    def quad_body(q):
        for u in range(4):
            proc(q * 4 + u, u, first=False, last=False)
    pl.loop(1, NCH // 4 - 1)(quad_body)

    for u in range(4):
        proc(NCH - 4 + u, u, first=False, last=(u >= 2))
    for u in (2, 3):
        wait_o(NCH - 4 + u, u)


def kernel(x, cumulative_skipped_flops, step_embeddings_weight):
    out = _sc_step_encoding(
        x.reshape(R, D),
        cumulative_skipped_flops.reshape(R),
        step_embeddings_weight,
    )
    return out.reshape(B, SEQ, D)
